# baseline (speedup 1.0000x reference)
import functools

import jax
import jax.numpy as jnp
from jax import lax
from jax.experimental import pallas as pl
from jax.experimental.pallas import tpu as pltpu
from jax.experimental.pallas import tpu_sc as plsc

_B, _S, _D, _T = 16, 512, 256, 2048
_NC, _NS = 2, 16
_HALF = _T // 2
_CH = 128
_NCH = _HALF // _CH
_L = 16
_SENT = 2**31 - 1


def _sc_expand_body(x_hbm, dur_hbm, out_hbm, tot_hbm,
                    dur_v, cum_v, mark_v, gidx_v, bufa_v, bufb_v, zbuf_v,
                    tot_v, gsem0, gsem1, ssem0, ssem1, zsem):
    cid = lax.axis_index("c")
    sid = lax.axis_index("s")
    wid = sid * _NC + cid
    b = wid // 2
    h = wid % 2
    iota = lax.iota(jnp.int32, _L)

    pltpu.sync_copy(dur_hbm.at[b], dur_v)

    def cum_body(i, carry):
        v = dur_v[pl.ds(i * _L, _L)]
        c = plsc.cumsum(v) + carry
        cum_v[pl.ds(i * _L, _L)] = c
        return c[_L - 1]

    total = lax.fori_loop(0, _S // _L, cum_body, jnp.int32(0))
    cum_v[pl.ds(_S, _L)] = jnp.full((_L,), _SENT, jnp.int32)

    @pl.when(h == (b % 2))
    def _():
        tot_v[...] = jnp.full((_L,), total, jnp.int32)
        pltpu.sync_copy(tot_v, tot_hbm.at[b])

    def zmark_body(i, _):
        mark_v[pl.ds(i * _L, _L)] = jnp.zeros((_L,), jnp.int32)
        return 0

    lax.fori_loop(0, _T // _L, zmark_body, 0)

    def zbuf_body(r, _):
        for k in range(_D // _L):
            zbuf_v[r, pl.ds(k * _L, _L)] = jnp.zeros((_L,), jnp.float32)
        return 0

    lax.fori_loop(0, _CH, zbuf_body, 0)

    def mark_body(i, _):
        cur = cum_v[pl.ds(i * _L, _L)]
        nxt = plsc.load_gather(cum_v, [i * _L + 1 + iota])
        msk = (cur != nxt) & (cur < _T)
        plsc.store_scatter(mark_v, [jnp.minimum(cur, _T - 1)],
                           i * _L + 1 + iota, mask=msk)
        return 0

    lax.fori_loop(0, _S // _L, mark_body, 0)

    def idx_body(i, carry):
        m = jnp.maximum(plsc.cummax(mark_v[pl.ds(i * _L, _L)]), carry)
        gidx_v[i // (_CH // _L), pl.ds((i % (_CH // _L)) * _L, _L)] = (
            b * _S + jnp.minimum(m, _S - 1))
        return m[_L - 1]

    lax.fori_loop(0, _T // _L, idx_body, jnp.int32(0))

    row0 = b * _T
    bufs = (bufa_v, bufb_v)
    gsems = (gsem0, gsem1)
    ssems = (ssem0, ssem1)
    cgs = [2 * c + h for c in range(_NCH)]
    starts = [cg * _CH for cg in cgs]
    preds = [total > s for s in starts]
    nvals = [jnp.clip(total - s, 0, _CH) for s in starts]
    gds, sds, zds = [], [], []
    for c in range(_NCH):
        slot = c % 2
        gds.append(pltpu.make_async_copy(
            x_hbm.at[gidx_v.at[cgs[c]]], bufs[slot], gsems[slot]))
        sds.append(pltpu.make_async_copy(
            bufs[slot], out_hbm.at[pl.ds(row0 + starts[c], _CH)],
            ssems[slot]))
        zds.append(pltpu.make_async_copy(
            zbuf_v, out_hbm.at[pl.ds(row0 + starts[c], _CH)], zsem))

    @pl.when(preds[0])
    def _():
        gds[0].start()

    for c in range(_NCH):
        if c + 1 < _NCH:
            if c >= 1:
                @pl.when(preds[c - 1])
                def _(c=c):
                    sds[c - 1].wait()

            @pl.when(preds[c + 1])
            def _(c=c):
                gds[c + 1].start()

        @pl.when(preds[c])
        def _(c=c):
            gds[c].wait()

            def zrow_body(r, _, buf=bufs[c % 2]):
                for k in range(_D // _L):
                    buf[r, pl.ds(k * _L, _L)] = jnp.zeros((_L,), jnp.float32)
                return 0

            lax.fori_loop(nvals[c], _CH, zrow_body, 0)
            sds[c].start()

        @pl.when(jnp.logical_not(preds[c]))
        def _(c=c):
            zds[c].start()

    for c in (_NCH - 2, _NCH - 1):
        @pl.when(preds[c])
        def _(c=c):
            sds[c].wait()

    for c in range(_NCH):
        @pl.when(jnp.logical_not(preds[c]))
        def _(c=c):
            zds[c].wait()


@functools.partial(
    pl.kernel,
    out_type=(jax.ShapeDtypeStruct((_B * _T, _D), jnp.float32),
              jax.ShapeDtypeStruct((_B, _L), jnp.int32)),
    mesh=plsc.VectorSubcoreMesh(core_axis_name="c", subcore_axis_name="s"),
    scratch_types=(
        pltpu.VMEM((_S,), jnp.int32),
        pltpu.VMEM((_S + _L,), jnp.int32),
        pltpu.VMEM((_T,), jnp.int32),
        pltpu.VMEM((_T // _CH, _CH), jnp.int32),
        pltpu.VMEM((_CH, _D), jnp.float32),
        pltpu.VMEM((_CH, _D), jnp.float32),
        pltpu.VMEM((_CH, _D), jnp.float32),
        pltpu.VMEM((_L,), jnp.int32),
        pltpu.SemaphoreType.DMA,
        pltpu.SemaphoreType.DMA,
        pltpu.SemaphoreType.DMA,
        pltpu.SemaphoreType.DMA,
        pltpu.SemaphoreType.DMA,
    ),
    compiler_params=pltpu.CompilerParams(needs_layout_passes=False),
)
def _sc_expand(x_hbm, dur_hbm, out_hbm, tot_hbm, *scratch):
    _sc_expand_body(x_hbm, dur_hbm, out_hbm, tot_hbm, *scratch)


def _ln(x, g, bb):
    m = jnp.mean(x, axis=-1, keepdims=True)
    v = jnp.mean((x - m) * (x - m), axis=-1, keepdims=True)
    return (x - m) * lax.rsqrt(v + 1e-5) * g + bb


def _pred_body(lip_ref, mask_ref, W1_ref, b1_ref, g1_ref, be1_ref,
               W2_ref, b2_ref, g2_ref, be2_ref, Wc_ref, bc_ref, out_ref):
    hm = lip_ref[0]
    a = lax.dot_general(hm, W1_ref[...], (((1,), (1,)), ((), ())),
                        preferred_element_type=jnp.float32) + b1_ref[...]
    a = jnp.maximum(a, 0.0)
    a = _ln(a, g1_ref[...], be1_ref[...])
    a = lax.dot_general(a, W2_ref[...], (((1,), (1,)), ((), ())),
                        preferred_element_type=jnp.float32) + b2_ref[...]
    a = jnp.maximum(a, 0.0)
    a = _ln(a, g2_ref[...], be2_ref[...])
    o = lax.dot_general(Wc_ref[...], a, (((1,), (1,)), ((), ())),
                        preferred_element_type=jnp.float32)
    o = o + bc_ref[0, 0]
    o = jnp.logaddexp(o, 0.0)
    out_ref[0] = o * (1.0 - mask_ref[0])


def _predictor(lip, mask_f, W1, b1, g1, be1, W2, b2, g2, be2, Wc, bc2):
    wspec2 = pl.BlockSpec((_D, _D), lambda i: (0, 0))
    vspec = pl.BlockSpec((_D,), lambda i: (0,))
    return pl.pallas_call(
        _pred_body,
        grid=(_B,),
        in_specs=[
            pl.BlockSpec((1, _S, _D), lambda i: (i, 0, 0)),
            pl.BlockSpec((1, 1, _S), lambda i: (i, 0, 0)),
            wspec2, vspec, vspec, vspec,
            wspec2, vspec, vspec, vspec,
            pl.BlockSpec((1, _D), lambda i: (0, 0)),
            pl.BlockSpec((1, _D), lambda i: (0, 0)),
        ],
        out_specs=pl.BlockSpec((1, 1, _S), lambda i: (i, 0, 0)),
        out_shape=jax.ShapeDtypeStruct((_B, 1, _S), jnp.float32),
    )(lip, mask_f, W1, b1, g1, be1, W2, b2, g2, be2, Wc, bc2)


def kernel(x, output_text_lip, src_mask, duration_target, max_len,
           W1, b1, g1, be1, W2, b2, g2, be2, Wc, bc):
    x_flat = x.reshape(_B * _S, _D)
    dur = duration_target.astype(jnp.int32)
    out_flat, totals = _sc_expand(x_flat, dur)

    mask_f = src_mask.astype(jnp.float32).reshape(_B, 1, _S)
    bc2 = jnp.broadcast_to(bc.reshape(1, 1), (1, _D))
    log_dur = _predictor(output_text_lip, mask_f, W1, b1, g1, be1,
                         W2, b2, g2, be2, Wc, bc2).reshape(_B, _S)
    x_expanded = out_flat.reshape(_B, _T, _D)
    mel_len = jnp.minimum(totals[:, 0], max_len)
    return x_expanded, log_dur, duration_target, mel_len


# speedup vs baseline: 1.0020x; 1.0020x over previous
import functools

import jax
import jax.numpy as jnp
from jax import lax
from jax.experimental import pallas as pl
from jax.experimental.pallas import tpu as pltpu
from jax.experimental.pallas import tpu_sc as plsc

_B, _S, _D, _T = 16, 512, 256, 2048
_NC, _NS = 2, 16
_HALF = _T // 2
_CH = 128
_NCH = _HALF // _CH
_L = 16
_SENT = 2**31 - 1


def _sc_expand_body(x_hbm, dur_hbm, out_hbm, tot_hbm,
                    dur_v, cum_v, mark_v, gidx_v, bufa_v, bufb_v, zbuf_v,
                    tot_v, gsem0, gsem1, ssem0, ssem1, zsem):
    cid = lax.axis_index("c")
    sid = lax.axis_index("s")
    wid = sid * _NC + cid
    b = wid // 2
    h = wid % 2
    iota = lax.iota(jnp.int32, _L)

    pltpu.sync_copy(dur_hbm.at[b], dur_v)

    def cum_body(i, carry):
        v = dur_v[pl.ds(i * _L, _L)]
        c = plsc.cumsum(v) + carry
        cum_v[pl.ds(i * _L, _L)] = c
        return c[_L - 1]

    total = lax.fori_loop(0, _S // _L, cum_body, jnp.int32(0))
    cum_v[pl.ds(_S, _L)] = jnp.full((_L,), _SENT, jnp.int32)

    @pl.when(h == (b % 2))
    def _():
        tot_v[...] = jnp.full((_L,), total, jnp.int32)
        pltpu.sync_copy(tot_v, tot_hbm.at[b])

    def zmark_body(i, _):
        mark_v[pl.ds(i * _L, _L)] = jnp.zeros((_L,), jnp.int32)
        return 0

    lax.fori_loop(0, _T // _L, zmark_body, 0)

    def zbuf_body(r, _):
        for k in range(_D // _L):
            zbuf_v[r, pl.ds(k * _L, _L)] = jnp.zeros((_L,), jnp.float32)
        return 0

    lax.fori_loop(0, _CH, zbuf_body, 0)

    def mark_body(i, _):
        cur = cum_v[pl.ds(i * _L, _L)]
        nxt = plsc.load_gather(cum_v, [i * _L + 1 + iota])
        msk = (cur != nxt) & (cur < _T)
        plsc.store_scatter(mark_v, [jnp.minimum(cur, _T - 1)],
                           i * _L + 1 + iota, mask=msk)
        return 0

    lax.fori_loop(0, _S // _L, mark_body, 0)

    def idx_body(i, carry):
        m = jnp.maximum(plsc.cummax(mark_v[pl.ds(i * _L, _L)]), carry)
        gidx_v[i // (_CH // _L), pl.ds((i % (_CH // _L)) * _L, _L)] = (
            b * _S + jnp.minimum(m, _S - 1))
        return m[_L - 1]

    lax.fori_loop(0, _T // _L, idx_body, jnp.int32(0))

    row0 = b * _T
    bufs = (bufa_v, bufb_v)
    gsems = (gsem0, gsem1)
    ssems = (ssem0, ssem1)
    par = jnp.bitwise_xor(h, b % 2)
    cgs = [2 * c + par for c in range(_NCH)]
    starts = [cg * _CH for cg in cgs]
    preds = [total > s for s in starts]
    nvals = [jnp.clip(total - s, 0, _CH) for s in starts]
    gds, sds, zds = [], [], []
    for c in range(_NCH):
        slot = c % 2
        gds.append(pltpu.make_async_copy(
            x_hbm.at[gidx_v.at[cgs[c]]], bufs[slot], gsems[slot]))
        sds.append(pltpu.make_async_copy(
            bufs[slot], out_hbm.at[pl.ds(row0 + starts[c], _CH)],
            ssems[slot]))
        zds.append(pltpu.make_async_copy(
            zbuf_v, out_hbm.at[pl.ds(row0 + starts[c], _CH)], zsem))

    @pl.when(preds[0])
    def _():
        gds[0].start()

    for c in range(_NCH):
        if c + 1 < _NCH:
            if c >= 1:
                @pl.when(preds[c - 1])
                def _(c=c):
                    sds[c - 1].wait()

            @pl.when(preds[c + 1])
            def _(c=c):
                gds[c + 1].start()

        @pl.when(preds[c])
        def _(c=c):
            gds[c].wait()

            def zrow_body(r, _, buf=bufs[c % 2]):
                for k in range(_D // _L):
                    buf[r, pl.ds(k * _L, _L)] = jnp.zeros((_L,), jnp.float32)
                return 0

            lax.fori_loop(nvals[c], _CH, zrow_body, 0)
            sds[c].start()

        @pl.when(jnp.logical_not(preds[c]))
        def _(c=c):
            zds[c].start()

    for c in (_NCH - 2, _NCH - 1):
        @pl.when(preds[c])
        def _(c=c):
            sds[c].wait()

    for c in range(_NCH):
        @pl.when(jnp.logical_not(preds[c]))
        def _(c=c):
            zds[c].wait()


@functools.partial(
    pl.kernel,
    out_type=(jax.ShapeDtypeStruct((_B * _T, _D), jnp.float32),
              jax.ShapeDtypeStruct((_B, _L), jnp.int32)),
    mesh=plsc.VectorSubcoreMesh(core_axis_name="c", subcore_axis_name="s"),
    scratch_types=(
        pltpu.VMEM((_S,), jnp.int32),
        pltpu.VMEM((_S + _L,), jnp.int32),
        pltpu.VMEM((_T,), jnp.int32),
        pltpu.VMEM((_T // _CH, _CH), jnp.int32),
        pltpu.VMEM((_CH, _D), jnp.float32),
        pltpu.VMEM((_CH, _D), jnp.float32),
        pltpu.VMEM((_CH, _D), jnp.float32),
        pltpu.VMEM((_L,), jnp.int32),
        pltpu.SemaphoreType.DMA,
        pltpu.SemaphoreType.DMA,
        pltpu.SemaphoreType.DMA,
        pltpu.SemaphoreType.DMA,
        pltpu.SemaphoreType.DMA,
    ),
    compiler_params=pltpu.CompilerParams(needs_layout_passes=False),
)
def _sc_expand(x_hbm, dur_hbm, out_hbm, tot_hbm, *scratch):
    _sc_expand_body(x_hbm, dur_hbm, out_hbm, tot_hbm, *scratch)


def _ln(x, g, bb):
    m = jnp.mean(x, axis=-1, keepdims=True)
    v = jnp.mean((x - m) * (x - m), axis=-1, keepdims=True)
    return (x - m) * lax.rsqrt(v + 1e-5) * g + bb


def _pred_body(lip_ref, mask_ref, W1_ref, b1_ref, g1_ref, be1_ref,
               W2_ref, b2_ref, g2_ref, be2_ref, Wc_ref, bc_ref, out_ref):
    hm = lip_ref[0]
    a = lax.dot_general(hm, W1_ref[...], (((1,), (1,)), ((), ())),
                        preferred_element_type=jnp.float32) + b1_ref[...]
    a = jnp.maximum(a, 0.0)
    a = _ln(a, g1_ref[...], be1_ref[...])
    a = lax.dot_general(a, W2_ref[...], (((1,), (1,)), ((), ())),
                        preferred_element_type=jnp.float32) + b2_ref[...]
    a = jnp.maximum(a, 0.0)
    a = _ln(a, g2_ref[...], be2_ref[...])
    o = lax.dot_general(Wc_ref[...], a, (((1,), (1,)), ((), ())),
                        preferred_element_type=jnp.float32)
    o = o + bc_ref[0, 0]
    o = jnp.logaddexp(o, 0.0)
    out_ref[0] = o * (1.0 - mask_ref[0])


def _predictor(lip, mask_f, W1, b1, g1, be1, W2, b2, g2, be2, Wc, bc2):
    wspec2 = pl.BlockSpec((_D, _D), lambda i: (0, 0))
    vspec = pl.BlockSpec((_D,), lambda i: (0,))
    return pl.pallas_call(
        _pred_body,
        grid=(_B,),
        in_specs=[
            pl.BlockSpec((1, _S, _D), lambda i: (i, 0, 0)),
            pl.BlockSpec((1, 1, _S), lambda i: (i, 0, 0)),
            wspec2, vspec, vspec, vspec,
            wspec2, vspec, vspec, vspec,
            pl.BlockSpec((1, _D), lambda i: (0, 0)),
            pl.BlockSpec((1, _D), lambda i: (0, 0)),
        ],
        out_specs=pl.BlockSpec((1, 1, _S), lambda i: (i, 0, 0)),
        out_shape=jax.ShapeDtypeStruct((_B, 1, _S), jnp.float32),
    )(lip, mask_f, W1, b1, g1, be1, W2, b2, g2, be2, Wc, bc2)


def kernel(x, output_text_lip, src_mask, duration_target, max_len,
           W1, b1, g1, be1, W2, b2, g2, be2, Wc, bc):
    x_flat = x.reshape(_B * _S, _D)
    dur = duration_target.astype(jnp.int32)
    out_flat, totals = _sc_expand(x_flat, dur)

    mask_f = src_mask.astype(jnp.float32).reshape(_B, 1, _S)
    bc2 = jnp.broadcast_to(bc.reshape(1, 1), (1, _D))
    log_dur = _predictor(output_text_lip, mask_f, W1, b1, g1, be1,
                         W2, b2, g2, be2, Wc, bc2).reshape(_B, _S)
    x_expanded = out_flat.reshape(_B, _T, _D)
    mel_len = jnp.minimum(totals[:, 0], max_len)
    return x_expanded, log_dur, duration_target, mel_len


# speedup vs baseline: 1.1326x; 1.1304x over previous
import functools

import jax
import jax.numpy as jnp
from jax import lax
from jax.experimental import pallas as pl
from jax.experimental.pallas import tpu as pltpu
from jax.experimental.pallas import tpu_sc as plsc

_B, _S, _D, _T = 16, 512, 256, 2048
_NC, _NS = 2, 16
_HALF = _T // 2
_CH = 128
_NCH = _HALF // _CH
_L = 16
_SENT = 2**31 - 1


def _sc_expand_body(x_hbm, dur_hbm, out_hbm, tot_hbm,
                    dur_v, cum_v, mark_v, gidx_v, bufa_v, bufb_v, zbuf_v,
                    tot_v, gsem0, gsem1, ssem0, ssem1, zsem):
    cid = lax.axis_index("c")
    sid = lax.axis_index("s")
    wid = sid * _NC + cid
    b = wid // 2
    h = wid % 2
    iota = lax.iota(jnp.int32, _L)

    pltpu.sync_copy(dur_hbm.at[b], dur_v)

    def cum_body(i, carry):
        v = dur_v[pl.ds(i * _L, _L)]
        c = plsc.cumsum(v) + carry
        cum_v[pl.ds(i * _L, _L)] = c
        return c[_L - 1]

    total = lax.fori_loop(0, _S // _L, cum_body, jnp.int32(0))
    cum_v[pl.ds(_S, _L)] = jnp.full((_L,), _SENT, jnp.int32)

    @pl.when(h == (b % 2))
    def _():
        tot_v[...] = jnp.full((_L,), total, jnp.int32)
        pltpu.sync_copy(tot_v, tot_hbm.at[b])

    row0 = b * _T
    bufs = (bufa_v, bufb_v)
    gsems = (gsem0, gsem1)
    ssems = (ssem0, ssem1)
    par = jnp.bitwise_xor(h, b % 2)
    cgs = [2 * c + par for c in range(_NCH)]
    starts = [cg * _CH for cg in cgs]
    preds = [total > s for s in starts]
    nvals = [jnp.clip(total - s, 0, _CH) for s in starts]
    gds, sds, zds = [], [], []
    for c in range(_NCH):
        slot = c % 2
        gds.append(pltpu.make_async_copy(
            x_hbm.at[gidx_v.at[cgs[c]]], bufs[slot], gsems[slot]))
        sds.append(pltpu.make_async_copy(
            bufs[slot], out_hbm.at[pl.ds(row0 + starts[c], _CH)],
            ssems[slot]))
        zds.append(pltpu.make_async_copy(
            zbuf_v, out_hbm.at[pl.ds(row0 + starts[c], _CH)], zsem))

    def zbuf_body(r, _):
        for k in range(_D // _L):
            zbuf_v[r, pl.ds(k * _L, _L)] = jnp.zeros((_L,), jnp.float32)
        return 0

    lax.fori_loop(0, _CH, zbuf_body, 0)

    for c in range(_NCH):
        @pl.when(jnp.logical_not(preds[c]))
        def _(c=c):
            zds[c].start()

    def zmark_body(i, _):
        mark_v[pl.ds(i * _L, _L)] = jnp.zeros((_L,), jnp.int32)
        return 0

    lax.fori_loop(0, _T // _L, zmark_body, 0)

    def mark_body(i, _):
        cur = cum_v[pl.ds(i * _L, _L)]
        nxt = plsc.load_gather(cum_v, [i * _L + 1 + iota])
        msk = (cur != nxt) & (cur < _T)
        plsc.store_scatter(mark_v, [jnp.minimum(cur, _T - 1)],
                           i * _L + 1 + iota, mask=msk)
        return 0

    lax.fori_loop(0, _S // _L, mark_body, 0)

    def idx_body(i, carry):
        m = jnp.maximum(plsc.cummax(mark_v[pl.ds(i * _L, _L)]), carry)
        gidx_v[i // (_CH // _L), pl.ds((i % (_CH // _L)) * _L, _L)] = (
            b * _S + jnp.minimum(m, _S - 1))
        return m[_L - 1]

    lax.fori_loop(0, _T // _L, idx_body, jnp.int32(0))

    @pl.when(preds[0])
    def _():
        gds[0].start()

    for c in range(_NCH):
        if c + 1 < _NCH:
            if c >= 1:
                @pl.when(preds[c - 1])
                def _(c=c):
                    sds[c - 1].wait()

            @pl.when(preds[c + 1])
            def _(c=c):
                gds[c + 1].start()

        @pl.when(preds[c])
        def _(c=c):
            gds[c].wait()

            def zrow_body(r, _, buf=bufs[c % 2]):
                for k in range(_D // _L):
                    buf[r, pl.ds(k * _L, _L)] = jnp.zeros((_L,), jnp.float32)
                return 0

            lax.fori_loop(nvals[c], _CH, zrow_body, 0)
            sds[c].start()

    for c in (_NCH - 2, _NCH - 1):
        @pl.when(preds[c])
        def _(c=c):
            sds[c].wait()

    for c in range(_NCH):
        @pl.when(jnp.logical_not(preds[c]))
        def _(c=c):
            zds[c].wait()


@functools.partial(
    pl.kernel,
    out_type=(jax.ShapeDtypeStruct((_B * _T, _D), jnp.float32),
              jax.ShapeDtypeStruct((_B, _L), jnp.int32)),
    mesh=plsc.VectorSubcoreMesh(core_axis_name="c", subcore_axis_name="s"),
    scratch_types=(
        pltpu.VMEM((_S,), jnp.int32),
        pltpu.VMEM((_S + _L,), jnp.int32),
        pltpu.VMEM((_T,), jnp.int32),
        pltpu.VMEM((_T // _CH, _CH), jnp.int32),
        pltpu.VMEM((_CH, _D), jnp.float32),
        pltpu.VMEM((_CH, _D), jnp.float32),
        pltpu.VMEM((_CH, _D), jnp.float32),
        pltpu.VMEM((_L,), jnp.int32),
        pltpu.SemaphoreType.DMA,
        pltpu.SemaphoreType.DMA,
        pltpu.SemaphoreType.DMA,
        pltpu.SemaphoreType.DMA,
        pltpu.SemaphoreType.DMA,
    ),
    compiler_params=pltpu.CompilerParams(needs_layout_passes=False),
)
def _sc_expand(x_hbm, dur_hbm, out_hbm, tot_hbm, *scratch):
    _sc_expand_body(x_hbm, dur_hbm, out_hbm, tot_hbm, *scratch)


def _ln(x, g, bb):
    m = jnp.mean(x, axis=-1, keepdims=True)
    v = jnp.mean((x - m) * (x - m), axis=-1, keepdims=True)
    return (x - m) * lax.rsqrt(v + 1e-5) * g + bb


def _pred_body(lip_ref, mask_ref, W1_ref, b1_ref, g1_ref, be1_ref,
               W2_ref, b2_ref, g2_ref, be2_ref, Wc_ref, bc_ref, out_ref):
    hm = lip_ref[0]
    a = lax.dot_general(hm, W1_ref[...], (((1,), (1,)), ((), ())),
                        preferred_element_type=jnp.float32) + b1_ref[...]
    a = jnp.maximum(a, 0.0)
    a = _ln(a, g1_ref[...], be1_ref[...])
    a = lax.dot_general(a, W2_ref[...], (((1,), (1,)), ((), ())),
                        preferred_element_type=jnp.float32) + b2_ref[...]
    a = jnp.maximum(a, 0.0)
    a = _ln(a, g2_ref[...], be2_ref[...])
    o = lax.dot_general(Wc_ref[...], a, (((1,), (1,)), ((), ())),
                        preferred_element_type=jnp.float32)
    o = o + bc_ref[0, 0]
    o = jnp.logaddexp(o, 0.0)
    out_ref[0] = o * (1.0 - mask_ref[0])


def _predictor(lip, mask_f, W1, b1, g1, be1, W2, b2, g2, be2, Wc, bc2):
    wspec2 = pl.BlockSpec((_D, _D), lambda i: (0, 0))
    vspec = pl.BlockSpec((_D,), lambda i: (0,))
    return pl.pallas_call(
        _pred_body,
        grid=(_B,),
        in_specs=[
            pl.BlockSpec((1, _S, _D), lambda i: (i, 0, 0)),
            pl.BlockSpec((1, 1, _S), lambda i: (i, 0, 0)),
            wspec2, vspec, vspec, vspec,
            wspec2, vspec, vspec, vspec,
            pl.BlockSpec((1, _D), lambda i: (0, 0)),
            pl.BlockSpec((1, _D), lambda i: (0, 0)),
        ],
        out_specs=pl.BlockSpec((1, 1, _S), lambda i: (i, 0, 0)),
        out_shape=jax.ShapeDtypeStruct((_B, 1, _S), jnp.float32),
    )(lip, mask_f, W1, b1, g1, be1, W2, b2, g2, be2, Wc, bc2)


def kernel(x, output_text_lip, src_mask, duration_target, max_len,
           W1, b1, g1, be1, W2, b2, g2, be2, Wc, bc):
    x_flat = x.reshape(_B * _S, _D)
    dur = duration_target.astype(jnp.int32)
    out_flat, totals = _sc_expand(x_flat, dur)

    mask_f = src_mask.astype(jnp.float32).reshape(_B, 1, _S)
    bc2 = jnp.broadcast_to(bc.reshape(1, 1), (1, _D))
    log_dur = _predictor(output_text_lip, mask_f, W1, b1, g1, be1,
                         W2, b2, g2, be2, Wc, bc2).reshape(_B, _S)
    x_expanded = out_flat.reshape(_B, _T, _D)
    mel_len = jnp.minimum(totals[:, 0], max_len)
    return x_expanded, log_dur, duration_target, mel_len
